# baseline (speedup 1.0000x reference)
import jax
import jax.numpy as jnp
from jax.experimental import pallas as pl
from jax.experimental.pallas import tpu as pltpu
from functools import partial

_BR1 = 200
_BR2 = 1000


def _l1_kernel(adj_ref, x_ref, w_ref, b_ref, h_ref, adjb_ref, u_ref):
    @pl.when(pl.program_id(0) == 0)
    def _():
        u_ref[...] = jnp.dot(
            x_ref[...].astype(jnp.bfloat16),
            w_ref[...].astype(jnp.bfloat16),
            preferred_element_type=jnp.float32,
        ).astype(jnp.bfloat16)

    a = adj_ref[...].astype(jnp.bfloat16)
    adjb_ref[...] = a
    acc = jnp.dot(a, u_ref[...], preferred_element_type=jnp.float32)
    acc = acc + b_ref[...]
    h_ref[...] = jnp.where(acc >= 0, acc, 0.01 * acc)


def _layer1(adj, x, w, b):
    n = adj.shape[0]
    f = w.shape[1]
    fin = x.shape[1]
    return pl.pallas_call(
        _l1_kernel,
        grid=(n // _BR1,),
        in_specs=[
            pl.BlockSpec((_BR1, n), lambda i: (i, 0)),
            pl.BlockSpec((n, fin), lambda i: (0, 0)),
            pl.BlockSpec((fin, f), lambda i: (0, 0)),
            pl.BlockSpec((1, f), lambda i: (0, 0)),
        ],
        out_specs=[
            pl.BlockSpec((_BR1, f), lambda i: (i, 0)),
            pl.BlockSpec((_BR1, n), lambda i: (i, 0)),
        ],
        out_shape=[
            jax.ShapeDtypeStruct((n, f), jnp.float32),
            jax.ShapeDtypeStruct((n, n), jnp.bfloat16),
        ],
        scratch_shapes=[pltpu.VMEM((n, f), jnp.bfloat16)],
    )(adj, x, w, b)


def _l23_kernel(adjb_ref, h1_ref, w2_ref, w3_ref, b2_ref, b3_ref, o_ref,
                u_ref, h2_ref):
    l = pl.program_id(0)
    i = pl.program_id(1)

    @pl.when((l == 0) & (i == 0))
    def _():
        u_ref[...] = jnp.dot(
            h1_ref[...].astype(jnp.bfloat16),
            w2_ref[...].astype(jnp.bfloat16),
            preferred_element_type=jnp.float32,
        ).astype(jnp.bfloat16)

    @pl.when((l == 1) & (i == 0))
    def _():
        u_ref[...] = jnp.dot(
            h2_ref[...].astype(jnp.bfloat16),
            w3_ref[...].astype(jnp.bfloat16),
            preferred_element_type=jnp.float32,
        ).astype(jnp.bfloat16)

    acc = jnp.dot(adjb_ref[...], u_ref[...], preferred_element_type=jnp.float32)
    v = acc + jnp.where(l == 0, b2_ref[...], b3_ref[...])
    v = jnp.where(l == 0, jnp.where(v >= 0, v, 0.01 * v), v)
    o_ref[...] = v

    @pl.when(l == 0)
    def _():
        h2_ref[pl.ds(i * _BR2, _BR2), :] = v


def _layer23(adjb, h1, w2, w3, b2, b3):
    n = adjb.shape[0]
    f = w2.shape[1]
    fin = h1.shape[1]
    return pl.pallas_call(
        _l23_kernel,
        grid=(2, n // _BR2),
        in_specs=[
            pl.BlockSpec((_BR2, n), lambda l, i: (i, 0)),
            pl.BlockSpec((n, fin), lambda l, i: (0, 0)),
            pl.BlockSpec((fin, f), lambda l, i: (0, 0)),
            pl.BlockSpec((f, f), lambda l, i: (0, 0)),
            pl.BlockSpec((1, f), lambda l, i: (0, 0)),
            pl.BlockSpec((1, f), lambda l, i: (0, 0)),
        ],
        out_specs=pl.BlockSpec((_BR2, f), lambda l, i: (i, 0)),
        out_shape=jax.ShapeDtypeStruct((n, f), jnp.float32),
        scratch_shapes=[
            pltpu.VMEM((n, f), jnp.bfloat16),
            pltpu.VMEM((n, f), jnp.float32),
        ],
    )(adjb, h1, w2, w3, b2, b3)


def kernel(x, adj, W1, b1, W2, b2, W3, b3):
    h1, adjb = _layer1(adj, x, W1, b1.reshape(1, -1))
    out = _layer23(adjb, h1, W2, W3, b2.reshape(1, -1), b3.reshape(1, -1))
    return out


# speedup vs baseline: 1.0027x; 1.0027x over previous
import jax
import jax.numpy as jnp
from jax.experimental import pallas as pl
from jax.experimental.pallas import tpu as pltpu
from functools import partial

_BR1 = 400
_BR2 = 1000


def _l1_kernel(adj_ref, x_ref, w_ref, b_ref, h_ref, adjb_ref, u_ref):
    @pl.when(pl.program_id(0) == 0)
    def _():
        u_ref[...] = jnp.dot(
            x_ref[...].astype(jnp.bfloat16),
            w_ref[...].astype(jnp.bfloat16),
            preferred_element_type=jnp.float32,
        ).astype(jnp.bfloat16)

    a = adj_ref[...].astype(jnp.bfloat16)
    adjb_ref[...] = a
    acc = jnp.dot(a, u_ref[...], preferred_element_type=jnp.float32)
    acc = acc + b_ref[...]
    h_ref[...] = jnp.where(acc >= 0, acc, 0.01 * acc)


def _layer1(adj, x, w, b):
    n = adj.shape[0]
    f = w.shape[1]
    fin = x.shape[1]
    return pl.pallas_call(
        _l1_kernel,
        grid=(n // _BR1,),
        in_specs=[
            pl.BlockSpec((_BR1, n), lambda i: (i, 0)),
            pl.BlockSpec((n, fin), lambda i: (0, 0)),
            pl.BlockSpec((fin, f), lambda i: (0, 0)),
            pl.BlockSpec((1, f), lambda i: (0, 0)),
        ],
        out_specs=[
            pl.BlockSpec((_BR1, f), lambda i: (i, 0)),
            pl.BlockSpec((_BR1, n), lambda i: (i, 0)),
        ],
        out_shape=[
            jax.ShapeDtypeStruct((n, f), jnp.float32),
            jax.ShapeDtypeStruct((n, n), jnp.bfloat16),
        ],
        scratch_shapes=[pltpu.VMEM((n, f), jnp.bfloat16)],
    )(adj, x, w, b)


def _l23_kernel(adjb_ref, h1_ref, w2_ref, w3_ref, b2_ref, b3_ref, o_ref,
                u_ref, h2_ref):
    l = pl.program_id(0)
    i = pl.program_id(1)

    @pl.when((l == 0) & (i == 0))
    def _():
        u_ref[...] = jnp.dot(
            h1_ref[...].astype(jnp.bfloat16),
            w2_ref[...].astype(jnp.bfloat16),
            preferred_element_type=jnp.float32,
        ).astype(jnp.bfloat16)

    @pl.when((l == 1) & (i == 0))
    def _():
        u_ref[...] = jnp.dot(
            h2_ref[...].astype(jnp.bfloat16),
            w3_ref[...].astype(jnp.bfloat16),
            preferred_element_type=jnp.float32,
        ).astype(jnp.bfloat16)

    acc = jnp.dot(adjb_ref[...], u_ref[...], preferred_element_type=jnp.float32)
    v = acc + jnp.where(l == 0, b2_ref[...], b3_ref[...])
    v = jnp.where(l == 0, jnp.where(v >= 0, v, 0.01 * v), v)
    o_ref[...] = v

    @pl.when(l == 0)
    def _():
        h2_ref[pl.ds(i * _BR2, _BR2), :] = v


def _layer23(adjb, h1, w2, w3, b2, b3):
    n = adjb.shape[0]
    f = w2.shape[1]
    fin = h1.shape[1]
    return pl.pallas_call(
        _l23_kernel,
        grid=(2, n // _BR2),
        in_specs=[
            pl.BlockSpec((_BR2, n), lambda l, i: (i, 0)),
            pl.BlockSpec((n, fin), lambda l, i: (0, 0)),
            pl.BlockSpec((fin, f), lambda l, i: (0, 0)),
            pl.BlockSpec((f, f), lambda l, i: (0, 0)),
            pl.BlockSpec((1, f), lambda l, i: (0, 0)),
            pl.BlockSpec((1, f), lambda l, i: (0, 0)),
        ],
        out_specs=pl.BlockSpec((_BR2, f), lambda l, i: (i, 0)),
        out_shape=jax.ShapeDtypeStruct((n, f), jnp.float32),
        scratch_shapes=[
            pltpu.VMEM((n, f), jnp.bfloat16),
            pltpu.VMEM((n, f), jnp.float32),
        ],
    )(adjb, h1, w2, w3, b2, b3)


def kernel(x, adj, W1, b1, W2, b2, W3, b3):
    h1, adjb = _layer1(adj, x, W1, b1.reshape(1, -1))
    out = _layer23(adjb, h1, W2, W3, b2.reshape(1, -1), b3.reshape(1, -1))
    return out
